# baseline (speedup 1.0000x reference)
import functools

import jax
import jax.numpy as jnp
import numpy as np
from jax.experimental import pallas as pl
from jax.experimental.pallas import tpu as pltpu
from jax.sharding import Mesh, PartitionSpec as P

_Q = 1024
_D = 16
_BK = 2000
_NKEYS = 100000


def _reduce_body(nsteps, y_ref, xt2_ref, m_ref, i_ref, m_scr, i_scr, xsq_scr):
    step = pl.program_id(0)

    @pl.when(step == 0)
    def _():
        xt2 = xt2_ref[...]
        xsq_scr[0, :] = jnp.sum(xt2 * xt2, axis=0) * 0.25

    y = y_ref[...]
    y_sq = jnp.sum(y * y, axis=1, keepdims=True)
    prod = jnp.dot(y, xt2_ref[...],
                   preferred_element_type=jnp.float32)
    d2 = (y_sq + xsq_scr[0, :][None, :]) + prod
    bm = jnp.min(d2, axis=0)
    ba = jnp.argmin(d2, axis=0).astype(jnp.int32)
    base = step * _BK

    @pl.when(step == 0)
    def _():
        m_scr[0, :] = bm
        i_scr[0, :] = ba

    @pl.when(step > 0)
    def _():
        cur_m = m_scr[0, :]
        better = bm < cur_m
        m_scr[0, :] = jnp.where(better, bm, cur_m)
        i_scr[0, :] = jnp.where(better, ba + base, i_scr[0, :])

    @pl.when(step == nsteps - 1)
    def _():
        m_ref[0, :] = m_scr[0, :]
        i_ref[0, :] = i_scr[0, :]


def _knn_reduce(y, xt2, *, interpret=False):
    nkeys = y.shape[0]
    nsteps = nkeys // _BK
    return pl.pallas_call(
        functools.partial(_reduce_body, nsteps),
        grid=(nsteps,),
        in_specs=[
            pl.BlockSpec((_BK, _D), lambda i: (i, 0)),
            pl.BlockSpec((_D, _Q), lambda i: (0, 0)),
        ],
        out_specs=[
            pl.BlockSpec((1, _Q), lambda i: (0, 0)),
            pl.BlockSpec((1, _Q), lambda i: (0, 0)),
        ],
        out_shape=[
            jax.ShapeDtypeStruct((1, _Q), jnp.float32),
            jax.ShapeDtypeStruct((1, _Q), jnp.int32),
        ],
        scratch_shapes=[
            pltpu.VMEM((1, _Q), jnp.float32),
            pltpu.VMEM((1, _Q), jnp.int32),
            pltpu.VMEM((1, _Q), jnp.float32),
        ],
        compiler_params=pltpu.CompilerParams(
            dimension_semantics=("arbitrary",),
        ),
        interpret=interpret,
    )(y, xt2)


def _sharded_reduce(y, xt2, n_shards, *, interpret=False):
    mesh = Mesh(np.array(jax.devices()[:n_shards]), ("k",))
    shard_keys = _NKEYS // n_shards

    def shard_fn(y_local, xt2_local):
        m, i = _knn_reduce(y_local, xt2_local, interpret=interpret)
        base = jax.lax.axis_index("k").astype(jnp.int32) * shard_keys
        return m, i + base

    m, i = jax.shard_map(
        shard_fn, mesh=mesh,
        in_specs=(P("k", None), P(None, None)),
        out_specs=(P("k", None), P("k", None)),
        check_vma=False,
    )(y, xt2)
    best = m[0] <= m[1]
    return (jnp.where(best, m[0], m[1])[None, :],
            jnp.where(best, i[0], i[1])[None, :])


def kernel(x, x_idx_start, y, y_idx_start, min_dists, nn_indices):
    xt2 = (-2.0 * x.reshape(_Q, _D)).T
    n_dev = len(jax.devices())
    if n_dev >= 2:
        m, i = _sharded_reduce(y, xt2, 2)
    else:
        m, i = _knn_reduce(y, xt2)
    old = jax.lax.dynamic_slice(min_dists, (x_idx_start,), (_Q,))
    new_d = jnp.sqrt(jnp.maximum(m.reshape(_Q), 0.0))
    upd_d = jnp.minimum(new_d, old)
    upd_i = (i.reshape(_Q) + y_idx_start).astype(nn_indices.dtype)
    min_dists_new = jax.lax.dynamic_update_slice(min_dists, upd_d,
                                                 (x_idx_start,))
    nn_indices_new = jax.lax.dynamic_update_slice(nn_indices, upd_i,
                                                  (x_idx_start,))
    return (min_dists_new, nn_indices_new)


# speedup vs baseline: 3.0808x; 3.0808x over previous
import functools

import jax
import jax.numpy as jnp
from jax.experimental import pallas as pl
from jax.experimental.pallas import tpu as pltpu

_Q = 1024
_D = 16
_BK = 4000
_NKEYS = 100000


def _reduce_body(nsteps, y_ref, xt2_ref, m_ref, i_ref, m_scr, i_scr, xsq_scr):
    step = pl.program_id(0)

    @pl.when(step == 0)
    def _():
        xt2 = xt2_ref[...]
        xsq_scr[0, :] = jnp.sum(xt2 * xt2, axis=0) * 0.25

    y = y_ref[...]
    y_sq = jnp.sum(y * y, axis=1, keepdims=True)
    prod = jnp.dot(y, xt2_ref[...],
                   preferred_element_type=jnp.float32)
    d2 = (y_sq + xsq_scr[0, :][None, :]) + prod
    bm = jnp.min(d2, axis=0)
    ba = jnp.argmin(d2, axis=0).astype(jnp.int32)
    base = step * _BK

    @pl.when(step == 0)
    def _():
        m_scr[0, :] = bm
        i_scr[0, :] = ba

    @pl.when(step > 0)
    def _():
        cur_m = m_scr[0, :]
        better = bm < cur_m
        m_scr[0, :] = jnp.where(better, bm, cur_m)
        i_scr[0, :] = jnp.where(better, ba + base, i_scr[0, :])

    @pl.when(step == nsteps - 1)
    def _():
        m_ref[0, :] = m_scr[0, :]
        i_ref[0, :] = i_scr[0, :]


def _knn_reduce(y, xt2, *, interpret=False):
    nkeys = y.shape[0]
    nsteps = nkeys // _BK
    return pl.pallas_call(
        functools.partial(_reduce_body, nsteps),
        grid=(nsteps,),
        in_specs=[
            pl.BlockSpec((_BK, _D), lambda i: (i, 0)),
            pl.BlockSpec((_D, _Q), lambda i: (0, 0)),
        ],
        out_specs=[
            pl.BlockSpec((1, _Q), lambda i: (0, 0)),
            pl.BlockSpec((1, _Q), lambda i: (0, 0)),
        ],
        out_shape=[
            jax.ShapeDtypeStruct((1, _Q), jnp.float32),
            jax.ShapeDtypeStruct((1, _Q), jnp.int32),
        ],
        scratch_shapes=[
            pltpu.VMEM((1, _Q), jnp.float32),
            pltpu.VMEM((1, _Q), jnp.int32),
            pltpu.VMEM((1, _Q), jnp.float32),
        ],
        compiler_params=pltpu.CompilerParams(
            dimension_semantics=("arbitrary",),
        ),
        interpret=interpret,
    )(y, xt2)


def kernel(x, x_idx_start, y, y_idx_start, min_dists, nn_indices):
    xt2 = (-2.0 * x.reshape(_Q, _D)).T
    m, i = _knn_reduce(y, xt2)
    old = jax.lax.dynamic_slice(min_dists, (x_idx_start,), (_Q,))
    new_d = jnp.sqrt(jnp.maximum(m.reshape(_Q), 0.0))
    upd_d = jnp.minimum(new_d, old)
    upd_i = (i.reshape(_Q) + y_idx_start).astype(nn_indices.dtype)
    min_dists_new = jax.lax.dynamic_update_slice(min_dists, upd_d,
                                                 (x_idx_start,))
    nn_indices_new = jax.lax.dynamic_update_slice(nn_indices, upd_i,
                                                  (x_idx_start,))
    return (min_dists_new, nn_indices_new)


# speedup vs baseline: 3.1019x; 1.0068x over previous
import functools

import jax
import jax.numpy as jnp
from jax.experimental import pallas as pl
from jax.experimental.pallas import tpu as pltpu

_Q = 1024
_D = 16
_BK = 5000
_NKEYS = 100000


def _reduce_body(nsteps, y_ref, xt2_ref, m_ref, i_ref, m_scr, i_scr, xsq_scr):
    step = pl.program_id(0)

    @pl.when(step == 0)
    def _():
        xt2 = xt2_ref[...]
        xsq_scr[0, :] = jnp.sum(xt2 * xt2, axis=0) * 0.25

    y = y_ref[...]
    y_sq = jnp.sum(y * y, axis=1, keepdims=True)
    prod = jnp.dot(y, xt2_ref[...],
                   preferred_element_type=jnp.float32)
    d2 = (y_sq + xsq_scr[0, :][None, :]) + prod
    bm = jnp.min(d2, axis=0)
    ba = jnp.argmin(d2, axis=0).astype(jnp.int32)
    base = step * _BK

    @pl.when(step == 0)
    def _():
        m_scr[0, :] = bm
        i_scr[0, :] = ba

    @pl.when(step > 0)
    def _():
        cur_m = m_scr[0, :]
        better = bm < cur_m
        m_scr[0, :] = jnp.where(better, bm, cur_m)
        i_scr[0, :] = jnp.where(better, ba + base, i_scr[0, :])

    @pl.when(step == nsteps - 1)
    def _():
        m_ref[0, :] = m_scr[0, :]
        i_ref[0, :] = i_scr[0, :]


def _knn_reduce(y, xt2, *, interpret=False):
    nkeys = y.shape[0]
    nsteps = nkeys // _BK
    return pl.pallas_call(
        functools.partial(_reduce_body, nsteps),
        grid=(nsteps,),
        in_specs=[
            pl.BlockSpec((_BK, _D), lambda i: (i, 0)),
            pl.BlockSpec((_D, _Q), lambda i: (0, 0)),
        ],
        out_specs=[
            pl.BlockSpec((1, _Q), lambda i: (0, 0)),
            pl.BlockSpec((1, _Q), lambda i: (0, 0)),
        ],
        out_shape=[
            jax.ShapeDtypeStruct((1, _Q), jnp.float32),
            jax.ShapeDtypeStruct((1, _Q), jnp.int32),
        ],
        scratch_shapes=[
            pltpu.VMEM((1, _Q), jnp.float32),
            pltpu.VMEM((1, _Q), jnp.int32),
            pltpu.VMEM((1, _Q), jnp.float32),
        ],
        compiler_params=pltpu.CompilerParams(
            dimension_semantics=("arbitrary",),
        ),
        interpret=interpret,
    )(y, xt2)


def kernel(x, x_idx_start, y, y_idx_start, min_dists, nn_indices):
    xt2 = (-2.0 * x.reshape(_Q, _D)).T
    m, i = _knn_reduce(y, xt2)
    old = jax.lax.dynamic_slice(min_dists, (x_idx_start,), (_Q,))
    new_d = jnp.sqrt(jnp.maximum(m.reshape(_Q), 0.0))
    upd_d = jnp.minimum(new_d, old)
    upd_i = (i.reshape(_Q) + y_idx_start).astype(nn_indices.dtype)
    min_dists_new = jax.lax.dynamic_update_slice(min_dists, upd_d,
                                                 (x_idx_start,))
    nn_indices_new = jax.lax.dynamic_update_slice(nn_indices, upd_i,
                                                  (x_idx_start,))
    return (min_dists_new, nn_indices_new)
